# baseline (speedup 1.0000x reference)
import functools

import jax
import jax.numpy as jnp
from jax import lax
from jax.experimental import pallas as pl
from jax.experimental.pallas import tpu as pltpu
from jax.experimental.pallas import tpu_sc as plsc

D_MODEL = 32
N_TOK = 16384
N_POS = 200
NUM_WORKERS = 32
BLK_TOK = 512
BLOCKS_PER_P = N_TOK // BLK_TOK
N_BLOCKS = N_POS * BLOCKS_PER_P
BLK_PER_W = N_BLOCKS // NUM_WORKERS
OUT_ELEMS = N_TOK * N_POS * D_MODEL
TBLK = 16 * 8 * 128

_mesh = plsc.VectorSubcoreMesh(core_axis_name="c", subcore_axis_name="s")


@functools.partial(
    pl.kernel,
    mesh=_mesh,
    out_type=jax.ShapeDtypeStruct((OUT_ELEMS,), jnp.float32),
    scratch_types=[
        pltpu.VMEM((BLK_TOK,), jnp.int32),
        pltpu.VMEM((BLK_TOK,), jnp.int32),
        pltpu.VMEM((BLK_TOK, D_MODEL), jnp.float32),
        pltpu.VMEM((BLK_TOK, D_MODEL), jnp.float32),
        pltpu.VMEM((TBLK,), jnp.float32),
        pltpu.VMEM((TBLK,), jnp.float32),
        pltpu.SemaphoreType.DMA,
        pltpu.SemaphoreType.DMA,
        pltpu.SemaphoreType.DMA,
        pltpu.SemaphoreType.DMA,
        pltpu.SemaphoreType.DMA,
        pltpu.SemaphoreType.DMA,
    ],
    compiler_params=pltpu.CompilerParams(use_tc_tiling_on_sc=False,
                                         needs_layout_passes=False),
)
def _emb_lookup(idx_hbm, table_hbm, out_hbm,
                i0, i1, g0, g1, t0, t1, si0, si1, sg0, sg1, so0, so1):
    wid = lax.axis_index("s") * 2 + lax.axis_index("c")
    bid0 = wid * BLK_PER_W
    iv = (i0, i1)
    gv = (g0, g1)
    tv = (t0, t1)
    si = (si0, si1)
    sg = (sg0, sg1)
    so = (so0, so1)

    def start_idx(n, b):
        bid = bid0 + n
        p = bid // BLOCKS_PER_P
        tc0 = (bid % BLOCKS_PER_P) * 4
        for k4 in range(4):
            pltpu.async_copy(idx_hbm.at[p // 8, tc0 + k4, p % 8],
                             iv[b].at[pl.ds(k4 * 128, 128)], si[b])

    def wait_idx(b):
        for k4 in range(4):
            pltpu.make_async_copy(idx_hbm.at[0, 0, 0],
                                  iv[b].at[pl.ds(0, 128)], si[b]).wait()

    def start_gather(b):
        pltpu.async_copy(table_hbm.at[iv[b]], gv[b], sg[b])

    def wait_gather(b):
        pltpu.make_async_copy(table_hbm.at[pl.ds(0, BLK_TOK)], gv[b],
                              sg[b]).wait()

    def transpose_block(b):
        @plsc.parallel_loop(0, 32, unroll=2)
        def _(c):
            lanes = lax.iota(jnp.int32, 16)
            rotf = (lanes + c) & 31
            dbase = ((rotf >> 3) << 12) + ((rotf & 7) << 7) + lanes
            zeros = lanes & 0
            lbase = (lanes << 5) + rotf
            for s in range(32):
                v = plsc.load_gather(gv[b], [zeros, lbase + (s * 512)])
                dst = dbase + ((s // 8) * 1024 + (s % 8) * 16)
                plsc.store_scatter(tv[b], [dst], v)

    def start_outs(n, b):
        bid = bid0 + n
        p = bid // BLOCKS_PER_P
        tc0 = (bid % BLOCKS_PER_P) * 4
        obase = (p * 512 + tc0) * 1024
        for a in range(4):
            pltpu.async_copy(tv[b].at[pl.ds(a * 4096, 4096)],
                             out_hbm.at[pl.ds(obase + a * 131072, 4096)],
                             so[b])

    def wait_outs(b):
        for a in range(4):
            pltpu.make_async_copy(tv[b].at[pl.ds(a * 4096, 4096)],
                                  out_hbm.at[pl.ds(0, 4096)],
                                  so[b]).wait()

    start_idx(0, 0)
    wait_idx(0)
    start_gather(0)
    start_idx(1, 1)
    wait_idx(1)
    start_gather(1)
    wait_gather(0)
    start_idx(2, 0)
    transpose_block(0)
    start_outs(0, 0)

    @pl.loop(0, (BLK_PER_W - 2) // 2)
    def _(mm):
        for db in (0, 1):
            n = 2 * mm + 2 + db
            pb = db
            wait_idx(pb)
            wait_outs(pb)
            start_gather(pb)
            wait_gather(1 - pb)

            @pl.when(n + 1 < BLK_PER_W)
            def _():
                start_idx(n + 1, 1 - pb)

            transpose_block(1 - pb)
            start_outs(n - 1, 1 - pb)

    wait_gather(1)
    transpose_block(1)
    start_outs(BLK_PER_W - 1, 1)
    wait_outs(0)
    wait_outs(1)


def kernel(x, emb_weight):
    x4 = x.T.reshape(25, 8, 128, 128).transpose(0, 2, 1, 3)
    out1 = _emb_lookup(x4, emb_weight)
    out5 = out1.reshape(N_POS, 4, 128, 8, 128)
    j = out5.transpose(0, 1, 3, 2, 4)
    k = j.reshape(N_POS, D_MODEL, N_TOK)
    return k.transpose(2, 0, 1)


# speedup vs baseline: 1.8418x; 1.8418x over previous
import functools

import jax
import jax.numpy as jnp
from jax import lax
from jax.experimental import pallas as pl
from jax.experimental.pallas import tpu as pltpu
from jax.experimental.pallas import tpu_sc as plsc

D_MODEL = 32
N_TOK = 16384
N_POS = 200
NUM_WORKERS = 32
BLK_TOK = 512
BLOCKS_PER_P = N_TOK // BLK_TOK
N_BLOCKS = N_POS * BLOCKS_PER_P
BLK_PER_W = N_BLOCKS // NUM_WORKERS
OUT_ELEMS = N_TOK * N_POS * D_MODEL
TBLK = 16 * 8 * 128

_mesh = plsc.VectorSubcoreMesh(core_axis_name="c", subcore_axis_name="s")


@functools.partial(
    pl.kernel,
    mesh=_mesh,
    out_type=jax.ShapeDtypeStruct((OUT_ELEMS,), jnp.float32),
    scratch_types=[
        pltpu.VMEM((BLK_TOK,), jnp.int32),
        pltpu.VMEM((BLK_TOK,), jnp.int32),
        pltpu.VMEM((BLK_TOK, D_MODEL), jnp.float32),
        pltpu.VMEM((BLK_TOK, D_MODEL), jnp.float32),
        pltpu.VMEM((TBLK,), jnp.float32),
        pltpu.VMEM((TBLK,), jnp.float32),
        pltpu.SemaphoreType.DMA,
        pltpu.SemaphoreType.DMA,
        pltpu.SemaphoreType.DMA,
        pltpu.SemaphoreType.DMA,
        pltpu.SemaphoreType.DMA,
        pltpu.SemaphoreType.DMA,
    ],
    compiler_params=pltpu.CompilerParams(use_tc_tiling_on_sc=False,
                                         needs_layout_passes=False),
)
def _emb_lookup(idx_hbm, table_hbm, out_hbm,
                i0, i1, g0, g1, t0, t1, si0, si1, sg0, sg1, so0, so1):
    wid = lax.axis_index("s") * 2 + lax.axis_index("c")
    bid0 = wid * BLK_PER_W
    iv = (i0, i1)
    gv = (g0, g1)
    tv = (t0, t1)
    si = (si0, si1)
    sg = (sg0, sg1)
    so = (so0, so1)

    def start_idx(n, b):
        bid = bid0 + n
        p = bid // BLOCKS_PER_P
        tc0 = (bid % BLOCKS_PER_P) * 4
        for k4 in range(4):
            pltpu.async_copy(idx_hbm.at[p // 8, tc0 + k4, p % 8],
                             iv[b].at[pl.ds(k4 * 128, 128)], si[b])

    def wait_idx(b):
        for k4 in range(4):
            pltpu.make_async_copy(idx_hbm.at[0, 0, 0],
                                  iv[b].at[pl.ds(0, 128)], si[b]).wait()

    def start_gather(b):
        pltpu.async_copy(table_hbm.at[iv[b]], gv[b], sg[b])

    def wait_gather(b):
        pltpu.make_async_copy(table_hbm.at[pl.ds(0, BLK_TOK)], gv[b],
                              sg[b]).wait()

    def transpose_block(b):
        @plsc.parallel_loop(0, 32, unroll=2)
        def _(c):
            lanes = lax.iota(jnp.int32, 16)
            rotf = (lanes + c) & 31
            dbase = ((rotf >> 3) << 12) + ((rotf & 7) << 7) + lanes
            for s in range(32):
                rows = lanes + (s * 16)
                v = plsc.load_gather(gv[b], [rows, rotf])
                dst = dbase + ((s // 8) * 1024 + (s % 8) * 16)
                plsc.store_scatter(tv[b], [dst], v)

    def start_outs(n, b):
        bid = bid0 + n
        p = bid // BLOCKS_PER_P
        tc0 = (bid % BLOCKS_PER_P) * 4
        obase = (p * 512 + tc0) * 1024
        for a in range(4):
            pltpu.async_copy(tv[b].at[pl.ds(a * 4096, 4096)],
                             out_hbm.at[pl.ds(obase + a * 131072, 4096)],
                             so[b])

    def wait_outs(b):
        for a in range(4):
            pltpu.make_async_copy(tv[b].at[pl.ds(a * 4096, 4096)],
                                  out_hbm.at[pl.ds(0, 4096)],
                                  so[b]).wait()

    start_idx(0, 0)
    wait_idx(0)
    start_gather(0)
    start_idx(1, 1)
    wait_idx(1)
    start_gather(1)
    wait_gather(0)
    start_idx(2, 0)
    transpose_block(0)
    start_outs(0, 0)

    @pl.loop(0, (BLK_PER_W - 2) // 2)
    def _(mm):
        for db in (0, 1):
            n = 2 * mm + 2 + db
            pb = db
            wait_idx(pb)
            wait_outs(pb)
            start_gather(pb)
            wait_gather(1 - pb)

            @pl.when(n + 1 < BLK_PER_W)
            def _():
                start_idx(n + 1, 1 - pb)

            transpose_block(1 - pb)
            start_outs(n - 1, 1 - pb)

    wait_gather(1)
    transpose_block(1)
    start_outs(BLK_PER_W - 1, 1)
    wait_outs(0)
    wait_outs(1)


def kernel(x, emb_weight):
    x4 = x.T.reshape(25, 8, 128, 128).transpose(0, 2, 1, 3)
    out1 = _emb_lookup(x4, emb_weight)
    out5 = out1.reshape(N_POS, 4, 128, 8, 128)
    j = out5.transpose(0, 1, 3, 2, 4)
    k = j.reshape(N_POS, D_MODEL, N_TOK)
    return k.transpose(2, 0, 1)
